# baseline (speedup 1.0000x reference)
import functools

import jax
import jax.numpy as jnp
from jax import lax
from jax.experimental import pallas as pl
from jax.experimental.pallas import tpu as pltpu
from jax.experimental.pallas import tpu_sc as plsc

N = 100000
H = 128
G = 512
C = 10
CP = 16

N_PAD = 100352
N_WORKERS = 16
CHUNK = N_PAD // N_WORKERS
IDX_ROWS = CHUNK // 128
IDX_ROWS_PAD = 56

MM_BLOCK = 3136


def _mm_body(x_ref, w_ref, y_ref):
    y = jnp.dot(x_ref[...], w_ref[...], preferred_element_type=jnp.float32)
    ones_col = (lax.broadcasted_iota(jnp.int32, (1, CP), 1) == C).astype(
        jnp.float32)
    y_ref[...] = y + ones_col


def _node_logits(x, w_pad):
    return pl.pallas_call(
        _mm_body,
        grid=(N_PAD // MM_BLOCK,),
        in_specs=[
            pl.BlockSpec((MM_BLOCK, H), lambda i: (i, 0)),
            pl.BlockSpec((H, CP), lambda i: (0, 0)),
        ],
        out_specs=pl.BlockSpec((MM_BLOCK, CP), lambda i: (i, 0)),
        out_shape=jax.ShapeDtypeStruct((N_PAD, CP), jnp.float32),
    )(x, w_pad)


ACC_ROWS = 640


def _sc_segment_body(y_hbm, idx_hbm, b_hbm, out_hbm,
                     y_v, idx_v, fin_v, zero_v, bvec_v, acc_sh,
                     sem_idx, sem_y, sem_b, sem_sc):
    c = lax.axis_index("c")
    s = lax.axis_index("s")

    for i in range(ACC_ROWS // N_WORKERS):
        zero_v[i, :] = jnp.zeros((CP,), jnp.float32)

    @pl.when(c == 0)
    def _():
        pltpu.sync_copy(zero_v, acc_sh.at[pl.ds(s * 40, 40), :])

    plsc.subcore_barrier()

    @pl.when(c == 0)
    def _():
        pltpu.sync_copy(idx_hbm.at[pl.ds(s * CHUNK, CHUNK)], idx_v)
        pltpu.sync_copy(y_hbm.at[pl.ds(s * CHUNK, CHUNK), :], y_v)
        pltpu.sync_copy(y_v, acc_sh.at[idx_v], add=True)

    plsc.subcore_barrier()

    @pl.when(c == 0)
    def _():
        pltpu.sync_copy(b_hbm, bvec_v)
        pltpu.sync_copy(acc_sh.at[pl.ds(s * 32, 32), :], fin_v)
        bv = bvec_v[...]
        cnt_lane = jnp.full((CP,), C, jnp.int32)
        for j in range(32):
            row = fin_v[j, :]
            cnt = row.at[cnt_lane].get(mode="promise_in_bounds")
            cnt = jnp.maximum(cnt, 1.0)
            fin_v[j, :] = row / cnt + bv
        pltpu.sync_copy(fin_v, out_hbm.at[pl.ds(s * 32, 32), :])


_sc_segment = functools.partial(
    pl.kernel,
    out_type=jax.ShapeDtypeStruct((G, CP), jnp.float32),
    mesh=plsc.VectorSubcoreMesh(core_axis_name="c", subcore_axis_name="s"),
    compiler_params=pltpu.CompilerParams(use_tc_tiling_on_sc=False),
    scratch_types=[
        pltpu.VMEM((CHUNK, CP), jnp.float32),
        pltpu.VMEM((CHUNK,), jnp.int32),
        pltpu.VMEM((32, CP), jnp.float32),
        pltpu.VMEM((ACC_ROWS // N_WORKERS, CP), jnp.float32),
        pltpu.VMEM((CP,), jnp.float32),
        pltpu.VMEM_SHARED((ACC_ROWS, CP), jnp.float32),
        pltpu.SemaphoreType.DMA,
        pltpu.SemaphoreType.DMA,
        pltpu.SemaphoreType.DMA,
        pltpu.SemaphoreType.DMA,
    ],
)(_sc_segment_body)


@jax.jit
def kernel(x, batch, W, b):
    w_pad = jnp.pad(W, ((0, 0), (0, CP - C)))
    b_pad = jnp.pad(b, (0, CP - C))
    batch_pad = jnp.concatenate(
        [batch.astype(jnp.int32),
         jnp.full((N_PAD - N,), G, jnp.int32)])
    y = _node_logits(x, w_pad)
    out = _sc_segment(y, batch_pad, b_pad)
    return out[:, :C]


# speedup vs baseline: 1.3718x; 1.3718x over previous
import functools

import jax
import jax.numpy as jnp
from jax import lax
from jax.experimental import pallas as pl
from jax.experimental.pallas import tpu as pltpu
from jax.experimental.pallas import tpu_sc as plsc

N = 100000
H = 128
G = 512
C = 10
CP = 16

N_PAD = 100352
N_WORKERS = 16
CHUNK = N_PAD // N_WORKERS
IDX_ROWS = CHUNK // 128
IDX_ROWS_PAD = 56

MM_BLOCK = 3136


def _mm_body(x_ref, w_ref, y_ref):
    y = jnp.dot(x_ref[...], w_ref[...], preferred_element_type=jnp.float32)
    ones_col = (lax.broadcasted_iota(jnp.int32, (1, CP), 1) == C).astype(
        jnp.float32)
    y = y + ones_col
    yv = y.reshape(MM_BLOCK // 8, 8, CP)
    y_ref[...] = jnp.concatenate([yv[:, g, :] for g in range(8)], axis=1)


def _node_logits(x, w_pad):
    return pl.pallas_call(
        _mm_body,
        grid=(N_PAD // MM_BLOCK,),
        in_specs=[
            pl.BlockSpec((MM_BLOCK, H), lambda i: (i, 0)),
            pl.BlockSpec((H, CP), lambda i: (0, 0)),
        ],
        out_specs=pl.BlockSpec((MM_BLOCK // 8, 8 * CP), lambda i: (i, 0)),
        out_shape=jax.ShapeDtypeStruct((N_PAD // 8, 8 * CP), jnp.float32),
    )(x, w_pad)


ACC_ROWS = 640


def _sc_segment_body(y_hbm, idx_hbm, b_hbm, out_hbm,
                     y_v, idx_v, fin_v, zero_v, bvec_v, acc_sh,
                     sem_idx, sem_y, sem_b, sem_sc):
    c = lax.axis_index("c")
    s = lax.axis_index("s")

    for i in range(ACC_ROWS // N_WORKERS):
        zero_v[i, :] = jnp.zeros((CP,), jnp.float32)

    @pl.when(c == 0)
    def _():
        pltpu.sync_copy(zero_v, acc_sh.at[pl.ds(s * 40, 40), :])

    plsc.subcore_barrier()

    @pl.when(c == 0)
    def _():
        pltpu.sync_copy(idx_hbm.at[pl.ds(s * CHUNK, CHUNK)], idx_v)
        pltpu.sync_copy(y_hbm.at[pl.ds(s * CHUNK, CHUNK), :], y_v)
        pltpu.sync_copy(y_v, acc_sh.at[idx_v], add=True)

    plsc.subcore_barrier()

    @pl.when(c == 0)
    def _():
        pltpu.sync_copy(b_hbm, bvec_v)
        pltpu.sync_copy(acc_sh.at[pl.ds(s * 32, 32), :], fin_v)
        bv = bvec_v[...]
        cnt_lane = jnp.full((CP,), C, jnp.int32)
        for j in range(32):
            row = fin_v[j, :]
            cnt = row.at[cnt_lane].get(mode="promise_in_bounds")
            cnt = jnp.maximum(cnt, 1.0)
            fin_v[j, :] = row / cnt + bv
        pltpu.sync_copy(fin_v, out_hbm.at[pl.ds(s * 32, 32), :])


_sc_segment = functools.partial(
    pl.kernel,
    out_type=jax.ShapeDtypeStruct((G, CP), jnp.float32),
    mesh=plsc.VectorSubcoreMesh(core_axis_name="c", subcore_axis_name="s"),
    compiler_params=pltpu.CompilerParams(use_tc_tiling_on_sc=False),
    scratch_types=[
        pltpu.VMEM((CHUNK, CP), jnp.float32),
        pltpu.VMEM((CHUNK,), jnp.int32),
        pltpu.VMEM((32, CP), jnp.float32),
        pltpu.VMEM((ACC_ROWS // N_WORKERS, CP), jnp.float32),
        pltpu.VMEM((CP,), jnp.float32),
        pltpu.VMEM_SHARED((ACC_ROWS, CP), jnp.float32),
        pltpu.SemaphoreType.DMA,
        pltpu.SemaphoreType.DMA,
        pltpu.SemaphoreType.DMA,
        pltpu.SemaphoreType.DMA,
    ],
)(_sc_segment_body)


@jax.jit
def kernel(x, batch, W, b):
    w_pad = jnp.pad(W, ((0, 0), (0, CP - C)))
    b_pad = jnp.pad(b, (0, CP - C))
    batch_pad = jnp.concatenate(
        [batch.astype(jnp.int32),
         jnp.full((N_PAD - N,), G, jnp.int32)])
    y = _node_logits(x, w_pad).reshape(N_PAD, CP)
    out = _sc_segment(y, batch_pad, b_pad)
    return out[:, :C]
